# baseline (speedup 1.0000x reference)
import jax
import jax.numpy as jnp
from jax import lax
from jax.experimental import pallas as pl
from jax.experimental.pallas import tpu as pltpu

_N = 50000
_S = 8192
_LANES = 128
_R = 392
_NPAD = _R * _LANES
_B = 49
_NCHAIN = 2

_NEG_INF = float("-inf")
_BIG = 2**31 - 1
_BIGF = float(2**25)


def _lex_combine(a, b):
    av, ax, ay, az = a
    bv, bx, by, bz = b
    take_a = av >= bv
    return (jnp.where(take_a, av, bv),
            jnp.where(take_a, ax, bx),
            jnp.where(take_a, ay, by),
            jnp.where(take_a, az, bz))


def _fps_body(xyz_ref, out_ref, dist_ref):
    row8 = lax.broadcasted_iota(jnp.int32, (8, _LANES), 0)
    col8 = lax.broadcasted_iota(jnp.int32, (8, _LANES), 1)
    base_iota = row8 * _LANES + col8
    base_iotaf = base_iota.astype(jnp.float32)
    lane_iota = lax.broadcasted_iota(jnp.int32, (1, _LANES), 1)

    row_iota = lax.broadcasted_iota(jnp.int32, (_R, _LANES), 0)
    col_iota = lax.broadcasted_iota(jnp.int32, (_R, _LANES), 1)
    fiota = row_iota * _LANES + col_iota
    dist_ref[:] = jnp.where(fiota < _N, jnp.inf, _NEG_INF)

    sel0 = lane_iota == 0
    px0 = jnp.sum(jnp.where(sel0, xyz_ref[0:1, :], 0.0),
                  axis=1, keepdims=True)
    py0 = jnp.sum(jnp.where(sel0, xyz_ref[_R:_R + 1, :], 0.0),
                  axis=1, keepdims=True)
    pz0 = jnp.sum(jnp.where(sel0, xyz_ref[2 * _R:2 * _R + 1, :], 0.0),
                  axis=1, keepdims=True)
    out_ref[pl.ds(0, 1), :] = jnp.where(lane_iota == 0, px0,
                                        jnp.where(lane_iota == 1, py0, pz0))

    bounds = [0, 25, _B]

    def body(i, carry):
        px, py, pz = carry
        chains = []
        for c in range(_NCHAIN):
            accv = jnp.full((8, _LANES), _NEG_INF, jnp.float32)
            accx = jnp.zeros((8, _LANES), jnp.float32)
            accy = jnp.zeros((8, _LANES), jnp.float32)
            accz = jnp.zeros((8, _LANES), jnp.float32)
            for b in range(bounds[c], bounds[c + 1]):
                r = 8 * b
                xb = xyz_ref[r:r + 8, :]
                yb = xyz_ref[_R + r:_R + r + 8, :]
                zb = xyz_ref[2 * _R + r:2 * _R + r + 8, :]
                d = ((xb - px) ** 2 + (zb - pz) ** 2) + (yb - py) ** 2
                ndb = jnp.minimum(dist_ref[r:r + 8, :], d)
                dist_ref[r:r + 8, :] = ndb
                better = ndb > accv
                accv = jnp.where(better, ndb, accv)
                accx = jnp.where(better, xb, accx)
                accy = jnp.where(better, yb, accy)
                accz = jnp.where(better, zb, accz)
            chains.append((accv, accx, accy, accz))
        acc = _lex_combine(chains[0], chains[1])
        for s in (4, 2, 1):
            rot = tuple(pltpu.roll(t, s, 0) for t in acc)
            acc = _lex_combine(acc, rot)
        val_c = acc[0][0:1, :]
        x_c = acc[1][0:1, :]
        y_c = acc[2][0:1, :]
        z_c = acc[3][0:1, :]
        mb = jnp.max(val_c, axis=1, keepdims=True)
        hit = val_c == mb
        sx = jnp.sum(jnp.where(hit, x_c, 0.0), axis=1, keepdims=True)
        sy = jnp.sum(jnp.where(hit, y_c, 0.0), axis=1, keepdims=True)
        sz = jnp.sum(jnp.where(hit, z_c, 0.0), axis=1, keepdims=True)
        nh_col = jnp.zeros((8, _LANES), jnp.float32)
        for b in range(_B):
            r = 8 * b
            nh_col = nh_col + jnp.where(dist_ref[r:r + 8, :] == mb, 1.0, 0.0)
        nh_srow = (nh_col[0:1, :] + nh_col[1:2, :] + nh_col[2:3, :]
                   + nh_col[3:4, :] + nh_col[4:5, :] + nh_col[5:6, :]
                   + nh_col[6:7, :] + nh_col[7:8, :])
        nh = jnp.sum(nh_srow, axis=1, keepdims=True)

        def tie_path(_):
            iwin = jnp.full((1, 1), _BIGF, jnp.float32)
            for b in range(_B):
                r = 8 * b
                eqb = dist_ref[r:r + 8, :] == mb
                cand = jnp.where(eqb, base_iotaf + float(r * _LANES), _BIGF)
                crow = jnp.min(cand, axis=0, keepdims=True)
                cmin = jnp.min(crow, axis=1, keepdims=True)
                iwin = jnp.minimum(iwin, cmin)
            tx = jnp.zeros((1, 1), jnp.float32)
            ty = jnp.zeros((1, 1), jnp.float32)
            tz = jnp.zeros((1, 1), jnp.float32)
            for b in range(_B):
                r = 8 * b
                wm = (base_iotaf + float(r * _LANES)) == iwin
                txc = jnp.where(wm, xyz_ref[r:r + 8, :], 0.0)
                tyc = jnp.where(wm, xyz_ref[_R + r:_R + r + 8, :], 0.0)
                tzc = jnp.where(wm, xyz_ref[2 * _R + r:2 * _R + r + 8, :], 0.0)
                tx = tx + jnp.sum(jnp.sum(txc, axis=0, keepdims=True),
                                  axis=1, keepdims=True)
                ty = ty + jnp.sum(jnp.sum(tyc, axis=0, keepdims=True),
                                  axis=1, keepdims=True)
                tz = tz + jnp.sum(jnp.sum(tzc, axis=0, keepdims=True),
                                  axis=1, keepdims=True)
            return tx, ty, tz

        wx, wy, wz = lax.cond(nh[0, 0] > 1.5, tie_path,
                              lambda _: (sx, sy, sz), 0)
        out_ref[pl.ds(i, 1), :] = jnp.where(lane_iota == 0, wx,
                                            jnp.where(lane_iota == 1, wy, wz))
        return (wx, wy, wz)

    lax.fori_loop(1, _S, body, (px0, py0, pz0))


def kernel(pos):
    posT = jnp.transpose(pos)
    padded = jnp.pad(posT, ((0, 0), (0, _NPAD - _N)))
    stacked = padded.reshape(3 * _R, _LANES)
    out = pl.pallas_call(
        _fps_body,
        out_shape=jax.ShapeDtypeStruct((_S, _LANES), jnp.float32),
        in_specs=[pl.BlockSpec(memory_space=pltpu.MemorySpace.VMEM)],
        out_specs=pl.BlockSpec(memory_space=pltpu.MemorySpace.VMEM),
        scratch_shapes=[pltpu.VMEM((_R, _LANES), jnp.float32)],
        interpret=False,
    )(stacked)
    return out[:, :3]


# speedup vs baseline: 1.1786x; 1.1786x over previous
import jax
import jax.numpy as jnp
from jax import lax
from jax.experimental import pallas as pl
from jax.experimental.pallas import tpu as pltpu

_N = 50000
_S = 8192
_LANES = 128
_R = 392
_NPAD = _R * _LANES
_B = 49
_NCHAIN = 2

_NEG_INF = float("-inf")
_BIG = 2**31 - 1
_BIGF = float(2**25)


def _lex_combine(a, b):
    av, ai, ax, ay, az = a
    bv, bi, bx, by, bz = b
    take_a = (av > bv) | ((av == bv) & (ai < bi))
    return (jnp.where(take_a, av, bv),
            jnp.where(take_a, ai, bi),
            jnp.where(take_a, ax, bx),
            jnp.where(take_a, ay, by),
            jnp.where(take_a, az, bz))


def _fps_body(xyz_ref, out_ref, dist_ref):
    row8 = lax.broadcasted_iota(jnp.int32, (8, _LANES), 0)
    col8 = lax.broadcasted_iota(jnp.int32, (8, _LANES), 1)
    base_iota = row8 * _LANES + col8
    base_iotaf = base_iota.astype(jnp.float32)
    lane_iota = lax.broadcasted_iota(jnp.int32, (1, _LANES), 1)

    row_iota = lax.broadcasted_iota(jnp.int32, (_R, _LANES), 0)
    col_iota = lax.broadcasted_iota(jnp.int32, (_R, _LANES), 1)
    fiota = row_iota * _LANES + col_iota
    dist_ref[:] = jnp.where(fiota < _N, jnp.inf, _NEG_INF)

    sel0 = lane_iota == 0
    px0 = jnp.sum(jnp.where(sel0, xyz_ref[0:1, :], 0.0),
                  axis=1, keepdims=True)
    py0 = jnp.sum(jnp.where(sel0, xyz_ref[_R:_R + 1, :], 0.0),
                  axis=1, keepdims=True)
    pz0 = jnp.sum(jnp.where(sel0, xyz_ref[2 * _R:2 * _R + 1, :], 0.0),
                  axis=1, keepdims=True)
    out_ref[pl.ds(0, 1), :] = jnp.where(lane_iota == 0, px0,
                                        jnp.where(lane_iota == 1, py0, pz0))

    bounds = [0, 25, _B]

    def body(i, carry):
        px, py, pz = carry
        chains = []
        for c in range(_NCHAIN):
            accv = jnp.full((8, _LANES), _NEG_INF, jnp.float32)
            acci = jnp.full((8, _LANES), _BIGF, jnp.float32)
            accx = jnp.zeros((8, _LANES), jnp.float32)
            accy = jnp.zeros((8, _LANES), jnp.float32)
            accz = jnp.zeros((8, _LANES), jnp.float32)
            for b in range(bounds[c], bounds[c + 1]):
                r = 8 * b
                xb = xyz_ref[r:r + 8, :]
                yb = xyz_ref[_R + r:_R + r + 8, :]
                zb = xyz_ref[2 * _R + r:2 * _R + r + 8, :]
                d = ((xb - px) ** 2 + (zb - pz) ** 2) + (yb - py) ** 2
                ndb = jnp.minimum(dist_ref[r:r + 8, :], d)
                dist_ref[r:r + 8, :] = ndb
                bidx = base_iotaf + float(r * _LANES)
                better = ndb > accv
                accv = jnp.where(better, ndb, accv)
                acci = jnp.where(better, bidx, acci)
                accx = jnp.where(better, xb, accx)
                accy = jnp.where(better, yb, accy)
                accz = jnp.where(better, zb, accz)
            chains.append((accv, acci, accx, accy, accz))
        acc = _lex_combine(chains[0], chains[1])
        for s in (4, 2, 1):
            rot = tuple(pltpu.roll(t, s, 0) for t in acc)
            acc = _lex_combine(acc, rot)
        val_c = acc[0][0:1, :]
        idx_c = acc[1][0:1, :]
        x_c = acc[2][0:1, :]
        y_c = acc[3][0:1, :]
        z_c = acc[4][0:1, :]
        mb = jnp.max(val_c, axis=1, keepdims=True)
        hit = val_c == mb
        sx = jnp.sum(jnp.where(hit, x_c, 0.0), axis=1, keepdims=True)
        sy = jnp.sum(jnp.where(hit, y_c, 0.0), axis=1, keepdims=True)
        sz = jnp.sum(jnp.where(hit, z_c, 0.0), axis=1, keepdims=True)
        nh = jnp.sum(jnp.where(hit, 1.0, 0.0), axis=1, keepdims=True)

        def tie_path(_):
            idxm = jnp.where(hit, idx_c, _BIGF)
            wi = jnp.min(idxm, axis=1, keepdims=True)
            wmask = idx_c == wi
            tx = jnp.sum(jnp.where(wmask, x_c, 0.0), axis=1, keepdims=True)
            ty = jnp.sum(jnp.where(wmask, y_c, 0.0), axis=1, keepdims=True)
            tz = jnp.sum(jnp.where(wmask, z_c, 0.0), axis=1, keepdims=True)
            return tx, ty, tz

        wx, wy, wz = lax.cond(nh[0, 0] > 1.5, tie_path,
                              lambda _: (sx, sy, sz), 0)
        out_ref[pl.ds(i, 1), :] = jnp.where(lane_iota == 0, wx,
                                            jnp.where(lane_iota == 1, wy, wz))
        return (wx, wy, wz)

    lax.fori_loop(1, _S, body, (px0, py0, pz0))


def kernel(pos):
    posT = jnp.transpose(pos)
    padded = jnp.pad(posT, ((0, 0), (0, _NPAD - _N)))
    stacked = padded.reshape(3 * _R, _LANES)
    out = pl.pallas_call(
        _fps_body,
        out_shape=jax.ShapeDtypeStruct((_S, _LANES), jnp.float32),
        in_specs=[pl.BlockSpec(memory_space=pltpu.MemorySpace.VMEM)],
        out_specs=pl.BlockSpec(memory_space=pltpu.MemorySpace.VMEM),
        scratch_shapes=[pltpu.VMEM((_R, _LANES), jnp.float32)],
        interpret=False,
    )(stacked)
    return out[:, :3]
